# Initial kernel scaffold; baseline (speedup 1.0000x reference)
import jax
import jax.numpy as jnp
from jax.experimental import pallas as pl


def kernel(real, imag, Lr0, Li0, Lr1, Li1, Lr2, Li2, Qreal, Qimag, R, W0, W1, W2, LW0, LW1, conv_w, conv_b):
    raise NotImplementedError("write your pallas kernel here")


# speedup vs baseline: 1.0104x; 1.0104x over previous
import jax
import jax.numpy as jnp
from jax import lax
from jax.experimental import pallas as pl
from jax.experimental.pallas import tpu as pltpu

_N = 2048
_C = 128
_M = 500
_BN = 256
_NB = _N // _BN
_F32 = jnp.float32
_BF16 = jnp.bfloat16


def _dot(a, b):
    return jnp.dot(a, b, preferred_element_type=_F32)


def _bf(x):
    return x.astype(_BF16)


def _pair_apply_body(l_ref, xr_ref, xi_ref, o_ref):
    xb = _bf(jnp.concatenate([xr_ref[...], xi_ref[...]], axis=1))
    o_ref[...] = _dot(l_ref[...], xb)


def _pair_apply(L_bf, Xr, Xi):
    return pl.pallas_call(
        _pair_apply_body,
        out_shape=jax.ShapeDtypeStruct((_N, 2 * _C), _F32),
    )(L_bf, Xr, Xi)


def _combine_body(arbi_ref, brai_ref, w_ref, xr_ref, xi_ref, or_ref, oi_ref,
                  *, residual_inside):
    Wb = _bf(w_ref[...])
    ar = _bf(arbi_ref[:, :_C])
    bi = _bf(arbi_ref[:, _C:])
    br = _bf(brai_ref[:, :_C])
    ai = _bf(brai_ref[:, _C:])
    Xr = xr_ref[...]
    Xi = xi_ref[...]
    rr = _dot(ar, Wb) - _dot(ai, Wb)
    ii = _dot(br, Wb) + _dot(bi, Wb)
    if residual_inside:
        rr = rr + Xr
        ii = ii + Xi
        m = (rr >= 0).astype(_F32)
        or_ref[...] = m * rr
        oi_ref[...] = m * ii
    else:
        m = (rr >= 0).astype(_F32)
        or_ref[...] = Xr + m * rr
        oi_ref[...] = Xi + m * ii


def _cheb_combine_body(arbi_ref, brai_ref, w_ref, xr_ref, xi_ref, or_ref,
                       oi_ref):
    _combine_body(arbi_ref, brai_ref, w_ref, xr_ref, xi_ref, or_ref, oi_ref,
                  residual_inside=True)


def _lanczos_combine_body(arbi_ref, brai_ref, w_ref, xr_ref, xi_ref, or_ref,
                          oi_ref):
    _combine_body(arbi_ref, brai_ref, w_ref, xr_ref, xi_ref, or_ref, oi_ref,
                  residual_inside=False)


def _combine(body, arbi, brai, W, Xr, Xi):
    return pl.pallas_call(
        body,
        out_shape=[
            jax.ShapeDtypeStruct((_N, _C), _F32),
            jax.ShapeDtypeStruct((_N, _C), _F32),
        ],
    )(arbi, brai, W, Xr, Xi)


def _layer(combine_body, Lr_bf, Li_bf, Xr, Xi, W):
    arbi = _pair_apply(Lr_bf, Xr, Xi)
    brai = _pair_apply(Li_bf, Xr, Xi)
    return _combine(combine_body, arbi, brai, W, Xr, Xi)


def _make_constr_body(sign):
    def body(qa_ref, qb_ref, qts_ref, t_ref, o_ref, scratch):
        k = pl.program_id(0)
        i = pl.program_id(1)
        T = t_ref[...]
        lhs = _bf(jnp.where(k == 0, qa_ref[...], qb_ref[...]) * T)
        d = _dot(lhs, qts_ref[0])
        @pl.when(k == 0)
        def _():
            scratch[pl.ds(i * _BN, _BN), :] = d
        @pl.when(k == 1)
        def _():
            o_ref[...] = _bf(scratch[pl.ds(i * _BN, _BN), :] + sign * d)
    return body


_constr_r_body = _make_constr_body(1.0)
_constr_i_body = _make_constr_body(-1.0)


def _constr_part(body, Qa, Qb, QTs, T):
    return pl.pallas_call(
        body,
        grid=(2, _NB),
        in_specs=[
            pl.BlockSpec((_BN, _M), lambda k, i: (i, 0)),
            pl.BlockSpec((_BN, _M), lambda k, i: (i, 0)),
            pl.BlockSpec((1, _M, _N), lambda k, i: (k, 0, 0)),
            pl.BlockSpec((1, _M), lambda k, i: (0, 0)),
        ],
        out_specs=pl.BlockSpec((_BN, _N), lambda k, i: (i, 0)),
        out_shape=jax.ShapeDtypeStruct((_N, _N), _BF16),
        scratch_shapes=[pltpu.VMEM((_N, _N), _F32)],
    )(Qa, Qb, QTs, T)


_DN_NT = (((1,), (1,)), ((), ()))


def _head_body(xr_ref, xi_ref, cw_ref, cb_ref, out_ref):
    Xr = _bf(xr_ref[...])
    Xi = _bf(xi_ref[...])
    cw = _bf(cw_ref[...])
    cb = cb_ref[...]
    yT = lax.dot_general(cw[:, :_C], Xr, _DN_NT, preferred_element_type=_F32)
    yT = yT + lax.dot_general(cw[:, _C:], Xi, _DN_NT,
                              preferred_element_type=_F32)
    yT = yT + cb
    mx = jnp.max(yT, axis=0, keepdims=True)
    sh = yT - mx
    out_ref[...] = sh - jnp.log(jnp.sum(jnp.exp(sh), axis=0, keepdims=True))


def _head(Xr, Xi, cw, cb):
    return pl.pallas_call(
        _head_body,
        out_shape=jax.ShapeDtypeStruct((2, _N), _F32),
    )(Xr, Xi, cw, cb)


@jax.jit
def kernel(real, imag, Lr0, Li0, Lr1, Li1, Lr2, Li2, Qreal, Qimag, R, W0,
           W1, W2, LW0, LW1, conv_w, conv_b):
    Xr, Xi = real, imag
    for Lr, Li, W in ((Lr0, Li0, W0), (Lr1, Li1, W1), (Lr2, Li2, W2)):
        Xr, Xi = _layer(_cheb_combine_body, _bf(Lr), _bf(Li), Xr, Xi, W)
    QTs = jnp.stack([Qreal.T.astype(_BF16), Qimag.T.astype(_BF16)])
    t10 = jnp.power(R, 10).reshape(1, _M)
    t20 = jnp.power(R, 20).reshape(1, _M)
    for T, LW in ((t10, LW0), (t20, LW1)):
        Lr_bf = _constr_part(_constr_r_body, Qreal, Qimag, QTs, T)
        Li_bf = _constr_part(_constr_i_body, Qimag, Qreal, QTs, T)
        Xr, Xi = _layer(_lanczos_combine_body, Lr_bf, Li_bf, Xr, Xi, LW)
    out = _head(Xr, Xi, conv_w, conv_b.reshape(2, 1))
    return out.reshape(1, 2, _N)
